# Initial kernel scaffold; baseline (speedup 1.0000x reference)
import jax
import jax.numpy as jnp
from jax.experimental import pallas as pl


def kernel(x, W, conv_w):
    raise NotImplementedError("write your pallas kernel here")


# speedup vs baseline: 9.5936x; 9.5936x over previous
import numpy as np
import jax
import jax.numpy as jnp
from jax.experimental import pallas as pl
from jax.experimental.pallas import tpu as pltpu

_B, _C, _H, _W = 2, 96, 64, 64
_P = _H * _W
_K = 16
_OUT_C = 96


def _segment_onehots():
    g = int(np.floor(np.sqrt(_K)))
    rows = (np.arange(_H) * g) // _H
    cols = (np.arange(_W) * g) // _W
    seg = (rows[:, None] * g + cols[None, :]).reshape(-1)
    sm = np.zeros((_P, _K), np.float32)
    sm[np.arange(_P), seg] = 1.0
    counts = sm.sum(axis=0)
    denom = counts + (counts == 0)
    sm_mean = sm / denom[None, :]
    sg = sm.T.copy()
    return sm_mean, sg


_SM_NP, _SG_NP = _segment_onehots()


def _dgfu_kernel(x_ref, w_ref, taps_ref, sm_ref, sg_ref, out_ref):
    x = x_ref[0]

    means_ck = jnp.dot(x, sm_ref[...], preferred_element_type=jnp.float32)
    means_kc = means_ck.T

    mw = jnp.dot(means_kc, w_ref[...], preferred_element_type=jnp.float32)
    sq = jnp.sum(mw * mw, axis=1, keepdims=True)
    gram = jnp.dot(mw, mw.T, preferred_element_type=jnp.float32)
    quad = sq + sq.T - 2.0 * gram
    row = jax.lax.broadcasted_iota(jnp.int32, (_K, _K), 0)
    col = jax.lax.broadcasted_iota(jnp.int32, (_K, _K), 1)
    adj = jnp.exp(-quad) * (row != col).astype(jnp.float32)

    am_kc = jnp.dot(adj, means_kc, preferred_element_type=jnp.float32)
    gathered = jnp.dot(am_kc.T, sg_ref[...], preferred_element_type=jnp.float32)
    feat = x + gathered

    pid = jax.lax.broadcasted_iota(jnp.int32, (1, _P), 1)
    wcol = pid & (_W - 1)
    hrow = pid >> 6
    edge = {
        (-1, 0): hrow != 0, (1, 0): hrow != (_H - 1),
        (0, -1): wcol != 0, (0, 1): wcol != (_W - 1),
    }
    acc = jnp.zeros((_OUT_C, _P), jnp.float32)
    t = 0
    for dy in (-1, 0, 1):
        for dx in (-1, 0, 1):
            o = _W * dy + dx
            shifted = feat if o == 0 else pltpu.roll(feat, -o % _P, axis=1)
            m = None
            for key in ((dy, 0), (0, dx)):
                if key in edge:
                    m = edge[key] if m is None else jnp.logical_and(m, edge[key])
            if m is not None:
                shifted = shifted * m.astype(jnp.float32)
            acc = acc + jnp.dot(taps_ref[t], shifted,
                                preferred_element_type=jnp.float32)
            t += 1
    out_ref[0] = acc


def kernel(x, W, conv_w):
    Bn, Cn, Hn, Wd = x.shape
    x_flat = x.reshape(Bn, Cn, Hn * Wd)
    taps = conv_w.transpose(2, 3, 0, 1).reshape(9, _OUT_C, _C)
    sm = jnp.asarray(_SM_NP)
    sg = jnp.asarray(_SG_NP)

    out_flat = pl.pallas_call(
        _dgfu_kernel,
        grid=(Bn,),
        in_specs=[
            pl.BlockSpec((1, _C, _P), lambda b: (b, 0, 0)),
            pl.BlockSpec((_C, _C), lambda b: (0, 0)),
            pl.BlockSpec((9, _OUT_C, _C), lambda b: (0, 0, 0)),
            pl.BlockSpec((_P, _K), lambda b: (0, 0)),
            pl.BlockSpec((_K, _P), lambda b: (0, 0)),
        ],
        out_specs=pl.BlockSpec((1, _OUT_C, _P), lambda b: (b, 0, 0)),
        out_shape=jax.ShapeDtypeStruct((Bn, _OUT_C, _P), jnp.float32),
        compiler_params=pltpu.CompilerParams(
            dimension_semantics=("parallel",),
        ),
    )(x_flat, W, taps, sm, sg)
    return out_flat.reshape(Bn, _OUT_C, Hn, Wd)


# speedup vs baseline: 10.1292x; 1.0558x over previous
import numpy as np
import jax
import jax.numpy as jnp
from jax.experimental import pallas as pl
from jax.experimental.pallas import tpu as pltpu

_B, _C, _H, _W = 2, 96, 64, 64
_P = _H * _W
_K = 16
_OUT_C = 96


def _segment_onehots():
    g = int(np.floor(np.sqrt(_K)))
    rows = (np.arange(_H) * g) // _H
    cols = (np.arange(_W) * g) // _W
    seg = (rows[:, None] * g + cols[None, :]).reshape(-1)
    sm = np.zeros((_P, _K), np.float32)
    sm[np.arange(_P), seg] = 1.0
    counts = sm.sum(axis=0)
    denom = counts + (counts == 0)
    sm_mean = sm / denom[None, :]
    sg = sm.T.copy()
    return sm_mean, sg


_SM_NP, _SG_NP = _segment_onehots()


def _dgfu_kernel(x_ref, w_ref, taps_ref, sm_ref, sg_ref, out_ref):
    x = x_ref[0]

    means_ck = jnp.dot(x, sm_ref[...], preferred_element_type=jnp.float32)
    means_kc = means_ck.T

    mw = jnp.dot(means_kc, w_ref[...], preferred_element_type=jnp.float32)
    sq = jnp.sum(mw * mw, axis=1, keepdims=True)
    gram = jnp.dot(mw, mw.T, preferred_element_type=jnp.float32)
    quad = sq + sq.T - 2.0 * gram
    row = jax.lax.broadcasted_iota(jnp.int32, (_K, _K), 0)
    col = jax.lax.broadcasted_iota(jnp.int32, (_K, _K), 1)
    adj = jnp.exp(-quad) * (row != col).astype(jnp.float32)

    am_kc = jnp.dot(adj, means_kc, preferred_element_type=jnp.float32)
    gathered = jnp.dot(am_kc.T, sg_ref[...], preferred_element_type=jnp.float32)
    feat = x + gathered

    pid = jax.lax.broadcasted_iota(jnp.int32, (1, _P), 1)
    wcol = pid & (_W - 1)
    hrow = pid >> 6
    m_top = (hrow != 0).astype(jnp.float32)
    m_bot = (hrow != (_H - 1)).astype(jnp.float32)
    m_lft = (wcol != 0).astype(jnp.float32)
    m_rgt = (wcol != (_W - 1)).astype(jnp.float32)

    v = {
        -1: pltpu.roll(feat, _W, axis=1) * m_top,
        0: feat,
        1: pltpu.roll(feat, _P - _W, axis=1) * m_bot,
    }
    acc = jnp.zeros((_OUT_C, _P), jnp.float32)
    for dx in (-1, 0, 1):
        g = jnp.zeros((_OUT_C, _P), jnp.float32)
        for dy in (-1, 0, 1):
            t = (dy + 1) * 3 + (dx + 1)
            g = g + jnp.dot(taps_ref[t], v[dy],
                            preferred_element_type=jnp.float32)
        if dx == -1:
            acc = acc + pltpu.roll(g, 1, axis=1) * m_lft
        elif dx == 1:
            acc = acc + pltpu.roll(g, _P - 1, axis=1) * m_rgt
        else:
            acc = acc + g
    out_ref[0] = acc


def kernel(x, W, conv_w):
    Bn, Cn, Hn, Wd = x.shape
    x_flat = x.reshape(Bn, Cn, Hn * Wd)
    taps = conv_w.transpose(2, 3, 0, 1).reshape(9, _OUT_C, _C)
    sm = jnp.asarray(_SM_NP)
    sg = jnp.asarray(_SG_NP)

    out_flat = pl.pallas_call(
        _dgfu_kernel,
        grid=(Bn,),
        in_specs=[
            pl.BlockSpec((1, _C, _P), lambda b: (b, 0, 0)),
            pl.BlockSpec((_C, _C), lambda b: (0, 0)),
            pl.BlockSpec((9, _OUT_C, _C), lambda b: (0, 0, 0)),
            pl.BlockSpec((_P, _K), lambda b: (0, 0)),
            pl.BlockSpec((_K, _P), lambda b: (0, 0)),
        ],
        out_specs=pl.BlockSpec((1, _OUT_C, _P), lambda b: (b, 0, 0)),
        out_shape=jax.ShapeDtypeStruct((Bn, _OUT_C, _P), jnp.float32),
        compiler_params=pltpu.CompilerParams(
            dimension_semantics=("parallel",),
        ),
    )(x_flat, W, taps, sm, sg)
    return out_flat.reshape(Bn, _OUT_C, Hn, Wd)


# speedup vs baseline: 10.2891x; 1.0158x over previous
import numpy as np
import jax
import jax.numpy as jnp
from jax.experimental import pallas as pl
from jax.experimental.pallas import tpu as pltpu

_B, _C, _H, _W = 2, 96, 64, 64
_P = _H * _W
_K = 16
_OUT_C = 96


def _segment_onehots():
    g = int(np.floor(np.sqrt(_K)))
    rows = (np.arange(_H) * g) // _H
    cols = (np.arange(_W) * g) // _W
    seg = (rows[:, None] * g + cols[None, :]).reshape(-1)
    sm = np.zeros((_P, _K), np.float32)
    sm[np.arange(_P), seg] = 1.0
    counts = sm.sum(axis=0)
    denom = counts + (counts == 0)
    sm_mean = sm / denom[None, :]
    sg = sm.T.copy()
    return sm_mean, sg


_SM_NP, _SG_NP = _segment_onehots()


def _dgfu_kernel(x_ref, w_ref, taps_ref, sm_ref, sg_ref, out_ref):
    x = x_ref[0]

    means_ck = jnp.dot(x, sm_ref[...], preferred_element_type=jnp.float32)
    means_kc = means_ck.T

    mw = jnp.dot(means_kc, w_ref[...], preferred_element_type=jnp.float32)
    sq = jnp.sum(mw * mw, axis=1, keepdims=True)
    gram = jnp.dot(mw, mw.T, preferred_element_type=jnp.float32)
    quad = sq + sq.T - 2.0 * gram
    row = jax.lax.broadcasted_iota(jnp.int32, (_K, _K), 0)
    col = jax.lax.broadcasted_iota(jnp.int32, (_K, _K), 1)
    adj = jnp.exp(-quad) * (row != col).astype(jnp.float32)

    am_kc = jnp.dot(adj, means_kc, preferred_element_type=jnp.float32)
    gathered = jnp.dot(am_kc.T, sg_ref[...], preferred_element_type=jnp.float32)
    feat = (x + gathered).astype(jnp.bfloat16)

    pid = jax.lax.broadcasted_iota(jnp.int32, (1, _P), 1)
    wcol = pid & (_W - 1)
    hrow = pid >> 6
    m_top = (hrow != 0).astype(jnp.bfloat16)
    m_bot = (hrow != (_H - 1)).astype(jnp.bfloat16)
    m_lft = (wcol != 0).astype(jnp.float32)
    m_rgt = (wcol != (_W - 1)).astype(jnp.float32)
    taps_bf = taps_ref[...].astype(jnp.bfloat16)

    v = {
        -1: pltpu.roll(feat, _W, axis=1) * m_top,
        0: feat,
        1: pltpu.roll(feat, _P - _W, axis=1) * m_bot,
    }
    acc = jnp.zeros((_OUT_C, _P), jnp.float32)
    for dx in (-1, 0, 1):
        g = jnp.zeros((_OUT_C, _P), jnp.float32)
        for dy in (-1, 0, 1):
            t = (dy + 1) * 3 + (dx + 1)
            g = g + jnp.dot(taps_bf[t], v[dy],
                            preferred_element_type=jnp.float32)
        if dx == -1:
            acc = acc + pltpu.roll(g, 1, axis=1) * m_lft
        elif dx == 1:
            acc = acc + pltpu.roll(g, _P - 1, axis=1) * m_rgt
        else:
            acc = acc + g
    out_ref[0] = acc


def kernel(x, W, conv_w):
    Bn, Cn, Hn, Wd = x.shape
    x_flat = x.reshape(Bn, Cn, Hn * Wd)
    taps = conv_w.transpose(2, 3, 0, 1).reshape(9, _OUT_C, _C)
    sm = jnp.asarray(_SM_NP)
    sg = jnp.asarray(_SG_NP)

    out_flat = pl.pallas_call(
        _dgfu_kernel,
        grid=(Bn,),
        in_specs=[
            pl.BlockSpec((1, _C, _P), lambda b: (b, 0, 0)),
            pl.BlockSpec((_C, _C), lambda b: (0, 0)),
            pl.BlockSpec((9, _OUT_C, _C), lambda b: (0, 0, 0)),
            pl.BlockSpec((_P, _K), lambda b: (0, 0)),
            pl.BlockSpec((_K, _P), lambda b: (0, 0)),
        ],
        out_specs=pl.BlockSpec((1, _OUT_C, _P), lambda b: (b, 0, 0)),
        out_shape=jax.ShapeDtypeStruct((Bn, _OUT_C, _P), jnp.float32),
        compiler_params=pltpu.CompilerParams(
            dimension_semantics=("parallel",),
        ),
    )(x_flat, W, taps, sm, sg)
    return out_flat.reshape(Bn, _OUT_C, Hn, Wd)
